# Initial kernel scaffold; baseline (speedup 1.0000x reference)
import jax
import jax.numpy as jnp
from jax.experimental import pallas as pl


def kernel(x, adj, edge_index, isVal, W0, b0, W1, b1, W2, b2, Wo, bo):
    raise NotImplementedError("write your pallas kernel here")


# speedup vs baseline: 5.3966x; 5.3966x over previous
import functools

import jax
import jax.numpy as jnp
from jax import lax
from jax.experimental import pallas as pl
from jax.experimental.pallas import tpu as pltpu
from jax.experimental.pallas import tpu_sc as plsc

N = 10000
E = 320000
NHID = 128
NCLASS = 64

_B = 80
_EPT = E // 32
_TPB = _EPT // _B
_CHUNK = 25
_NCHUNK = _TPB // _CHUNK
_NPAD = 10240
_STRIPE = _NPAD // 16
_ZROWS = 64

_DN = lax.GatherDimensionNumbers(
    offset_dims=(), collapsed_slice_dims=(0,), start_index_map=(0,))


def _make_edge_agg(d):
  mesh = plsc.VectorSubcoreMesh(core_axis_name="c", subcore_axis_name="s")

  @functools.partial(
      pl.kernel,
      mesh=mesh,
      out_type=jax.ShapeDtypeStruct((2, _NPAD, d), jnp.float32),
      scratch_types=[
          pltpu.VMEM((_CHUNK, _B), jnp.int32),
          pltpu.VMEM((_CHUNK, _B), jnp.int32),
          pltpu.VMEM((_CHUNK * _B,), jnp.float32),
          pltpu.VMEM((_B, d), jnp.float32),
          pltpu.VMEM((_ZROWS, d), jnp.float32),
          pltpu.VMEM_SHARED((_NPAD, d), jnp.float32),
          pltpu.SemaphoreType.DMA,
      ],
  )
  def edge_agg(s_hbm, src_hbm, dst_hbm, adj_hbm, out_hbm,
               src_v, dst_v, adj_v, rows_v, zero_v, acc, sem):
    c = lax.axis_index("c")
    sid = lax.axis_index("s")
    w = c * 16 + sid

    def zrow(i, carry):
      for k in range(d // 16):
        zero_v[i, pl.ds(k * 16, 16)] = jnp.zeros((16,), jnp.float32)
      return carry

    lax.fori_loop(0, _ZROWS, zrow, 0)
    for q in range(_STRIPE // _ZROWS):
      pltpu.sync_copy(zero_v, acc.at[pl.ds(sid * _STRIPE + q * _ZROWS, _ZROWS)])
    plsc.subcore_barrier()

    def chunk(q, carry):
      m = w * _NCHUNK + q
      pltpu.sync_copy(src_hbm.at[m], src_v)
      pltpu.sync_copy(dst_hbm.at[m], dst_v)
      pltpu.sync_copy(adj_hbm.at[pl.ds(m * (_CHUNK * _B), _CHUNK * _B)], adj_v)

      def batch(j, carry2):
        pltpu.async_copy(s_hbm.at[src_v.at[j]], rows_v, sem).wait()

        def scale(g, c2):
          va = adj_v[pl.ds(j * _B + g * 16, 16)]
          for l in range(16):
            e = g * 16 + l
            a = lax.gather(va, jnp.full((16, 1), l, jnp.int32), _DN,
                           slice_sizes=(1,),
                           mode=lax.GatherScatterMode.PROMISE_IN_BOUNDS)
            for k in range(d // 16):
              rows_v[e, pl.ds(k * 16, 16)] = rows_v[e, pl.ds(k * 16, 16)] * a
          return c2

        lax.fori_loop(0, _B // 16, scale, 0)
        pltpu.sync_copy(rows_v, acc.at[dst_v.at[j]], add=True)
        return carry2

      lax.fori_loop(0, _CHUNK, batch, 0)
      return carry

    lax.fori_loop(0, _NCHUNK, chunk, 0)
    plsc.subcore_barrier()
    pltpu.sync_copy(acc.at[pl.ds(sid * _STRIPE, _STRIPE)],
                    out_hbm.at[c].at[pl.ds(sid * _STRIPE, _STRIPE)])

  return edge_agg


_edge_agg = _make_edge_agg(NHID)

_ROWS = 1000
_GRID = N // _ROWS


def _mm_body(x_ref, w_ref, o_ref):
  o_ref[...] = jnp.dot(x_ref[...], w_ref[...],
                       preferred_element_type=jnp.float32)


def _tc_matmul(x, w):
  f, k = w.shape
  return pl.pallas_call(
      _mm_body,
      grid=(_GRID,),
      in_specs=[
          pl.BlockSpec((_ROWS, f), lambda i: (i, 0)),
          pl.BlockSpec((f, k), lambda i: (0, 0)),
      ],
      out_specs=pl.BlockSpec((_ROWS, k), lambda i: (i, 0)),
      out_shape=jax.ShapeDtypeStruct((N, k), jnp.float32),
  )(x, w)


def _combine_mm_body(p_ref, b_ref, w_ref, o_ref):
  h = jnp.maximum(p_ref[0] + p_ref[1] + b_ref[...], 0.0)
  o_ref[...] = jnp.dot(h, w_ref[...], preferred_element_type=jnp.float32)


def _tc_combine_matmul(p, b, w):
  f, k = w.shape
  return pl.pallas_call(
      _combine_mm_body,
      grid=(_GRID,),
      in_specs=[
          pl.BlockSpec((2, _ROWS, f), lambda i: (0, i, 0)),
          pl.BlockSpec((1, f), lambda i: (0, 0)),
          pl.BlockSpec((f, k), lambda i: (0, 0)),
      ],
      out_specs=pl.BlockSpec((_ROWS, k), lambda i: (i, 0)),
      out_shape=jax.ShapeDtypeStruct((N, k), jnp.float32),
  )(p, b, w)


def _final_body(p_ref, b_ref, o_ref):
  o_ref[...] = (p_ref[0, :, :NCLASS] + p_ref[1, :, :NCLASS] + b_ref[...])


def _tc_final(p, b):
  return pl.pallas_call(
      _final_body,
      grid=(_GRID,),
      in_specs=[
          pl.BlockSpec((2, _ROWS, NHID), lambda i: (0, i, 0)),
          pl.BlockSpec((1, NCLASS), lambda i: (0, 0)),
      ],
      out_specs=pl.BlockSpec((_ROWS, NCLASS), lambda i: (i, 0)),
      out_shape=jax.ShapeDtypeStruct((N, NCLASS), jnp.float32),
  )(p, b)


def _agg(s, src3, dst3, adj):
  return _edge_agg(s, src3, dst3, adj)[:, :N, :]


def kernel(x, adj, edge_index, isVal, W0, b0, W1, b1, W2, b2, Wo, bo):
  del isVal
  src3 = edge_index[0].reshape(32 * _NCHUNK, _CHUNK, _B)
  dst3 = edge_index[1].reshape(32 * _NCHUNK, _CHUNK, _B)
  wo_pad = jnp.pad(Wo, ((0, 0), (0, NHID - NCLASS)))

  s = _tc_matmul(x, W0)
  p = _agg(s, src3, dst3, adj)
  s = _tc_combine_matmul(p, b0.reshape(1, NHID), W1)
  p = _agg(s, src3, dst3, adj)
  s = _tc_combine_matmul(p, b1.reshape(1, NHID), W2)
  p = _agg(s, src3, dst3, adj)
  s = _tc_combine_matmul(p, b2.reshape(1, NHID), wo_pad)
  p = _agg(s, src3, dst3, adj)
  return _tc_final(p, bo.reshape(1, NCLASS))


# speedup vs baseline: 8.0840x; 1.4980x over previous
import functools

import jax
import jax.numpy as jnp
from jax import lax
from jax.experimental import pallas as pl
from jax.experimental.pallas import tpu as pltpu
from jax.experimental.pallas import tpu_sc as plsc

N = 10000
E = 320000
NHID = 128
NCLASS = 64

_B = 80
_EPT = E // 32
_TPB = _EPT // _B
_CHUNK = 25
_NCHUNK = _TPB // _CHUNK
_NPAD = 10240
_STRIPE = _NPAD // 16
_ZROWS = 64

_DN = lax.GatherDimensionNumbers(
    offset_dims=(), collapsed_slice_dims=(0,), start_index_map=(0,))


def _make_edge_agg(d):
  mesh = plsc.VectorSubcoreMesh(core_axis_name="c", subcore_axis_name="s")

  @functools.partial(
      pl.kernel,
      mesh=mesh,
      out_type=jax.ShapeDtypeStruct((2, _NPAD, d), jnp.float32),
      scratch_types=[
          pltpu.VMEM((_CHUNK, _B), jnp.int32),
          pltpu.VMEM((_CHUNK, _B), jnp.int32),
          pltpu.VMEM((_CHUNK * _B,), jnp.float32),
          pltpu.VMEM((_B, d), jnp.float32),
          pltpu.VMEM((_B, d), jnp.float32),
          pltpu.VMEM((_ZROWS, d), jnp.float32),
          pltpu.VMEM_SHARED((_NPAD, d), jnp.float32),
          pltpu.SemaphoreType.DMA,
          pltpu.SemaphoreType.DMA,
      ],
  )
  def edge_agg(s_hbm, src_hbm, dst_hbm, adj_hbm, out_hbm,
               src_v, dst_v, adj_v, rows0_v, rows1_v, zero_v, acc,
               gsem, ssem):
    c = lax.axis_index("c")
    sid = lax.axis_index("s")
    w = c * 16 + sid
    rows = (rows0_v, rows1_v)

    def zrow(i, carry):
      for k in range(d // 16):
        zero_v[i, pl.ds(k * 16, 16)] = jnp.zeros((16,), jnp.float32)
      return carry

    lax.fori_loop(0, _ZROWS, zrow, 0)
    for q in range(_STRIPE // _ZROWS):
      pltpu.async_copy(
          zero_v, acc.at[pl.ds(sid * _STRIPE + q * _ZROWS, _ZROWS)], gsem)
    for q in range(_STRIPE // _ZROWS):
      pltpu.make_async_copy(
          zero_v, acc.at[pl.ds(sid * _STRIPE + q * _ZROWS, _ZROWS)],
          gsem).wait()
    plsc.subcore_barrier()

    def scale_batch(buf, j):
      def scale(g, c2):
        va = adj_v[pl.ds(j * _B + g * 16, 16)]
        for l in range(16):
          e = g * 16 + l
          a = lax.gather(va, jnp.full((16, 1), l, jnp.int32), _DN,
                         slice_sizes=(1,),
                         mode=lax.GatherScatterMode.PROMISE_IN_BOUNDS)
          for k in range(d // 16):
            buf[e, pl.ds(k * 16, 16)] = buf[e, pl.ds(k * 16, 16)] * a
        return c2

      lax.fori_loop(0, _B // 16, scale, 0)

    def chunk(q, carry):
      m = w * _NCHUNK + q
      pltpu.async_copy(src_hbm.at[m], src_v, gsem)
      pltpu.async_copy(dst_hbm.at[m], dst_v, gsem)
      pltpu.async_copy(adj_hbm.at[pl.ds(m * (_CHUNK * _B), _CHUNK * _B)],
                       adj_v, gsem)
      pltpu.make_async_copy(src_hbm.at[m], src_v, gsem).wait()
      pltpu.make_async_copy(dst_hbm.at[m], dst_v, gsem).wait()
      pltpu.make_async_copy(adj_hbm.at[pl.ds(0, _CHUNK * _B)],
                            adj_v, gsem).wait()
      pltpu.async_copy(s_hbm.at[src_v.at[0]], rows[0], gsem)
      for j in range(_CHUNK):
        cur = rows[j % 2]
        pltpu.make_async_copy(s_hbm.at[src_v.at[j]], cur, gsem).wait()
        if j + 1 < _CHUNK:
          if j >= 1:
            pltpu.make_async_copy(cur, acc.at[dst_v.at[j]], ssem).wait()
          pltpu.async_copy(s_hbm.at[src_v.at[j + 1]], rows[(j + 1) % 2], gsem)
        scale_batch(cur, j)
        pltpu.async_copy(cur, acc.at[dst_v.at[j]], ssem, add=True)
      pltpu.make_async_copy(rows[0], acc.at[dst_v.at[0]], ssem).wait()
      pltpu.make_async_copy(rows[0], acc.at[dst_v.at[0]], ssem).wait()
      return carry

    lax.fori_loop(0, _NCHUNK, chunk, 0)
    plsc.subcore_barrier()
    pltpu.sync_copy(acc.at[pl.ds(sid * _STRIPE, _STRIPE)],
                    out_hbm.at[c].at[pl.ds(sid * _STRIPE, _STRIPE)])

  return edge_agg


_edge_agg = _make_edge_agg(NHID)

_ROWS = 1000
_GRID = N // _ROWS


def _mm_body(x_ref, w_ref, o_ref):
  o_ref[...] = jnp.dot(x_ref[...], w_ref[...],
                       preferred_element_type=jnp.float32)


def _tc_matmul(x, w):
  f, k = w.shape
  return pl.pallas_call(
      _mm_body,
      grid=(_GRID,),
      in_specs=[
          pl.BlockSpec((_ROWS, f), lambda i: (i, 0)),
          pl.BlockSpec((f, k), lambda i: (0, 0)),
      ],
      out_specs=pl.BlockSpec((_ROWS, k), lambda i: (i, 0)),
      out_shape=jax.ShapeDtypeStruct((N, k), jnp.float32),
  )(x, w)


def _combine_mm_body(p_ref, b_ref, w_ref, o_ref):
  h = jnp.maximum(p_ref[0] + p_ref[1] + b_ref[...], 0.0)
  o_ref[...] = jnp.dot(h, w_ref[...], preferred_element_type=jnp.float32)


def _tc_combine_matmul(p, b, w):
  f, k = w.shape
  return pl.pallas_call(
      _combine_mm_body,
      grid=(_GRID,),
      in_specs=[
          pl.BlockSpec((2, _ROWS, f), lambda i: (0, i, 0)),
          pl.BlockSpec((1, f), lambda i: (0, 0)),
          pl.BlockSpec((f, k), lambda i: (0, 0)),
      ],
      out_specs=pl.BlockSpec((_ROWS, k), lambda i: (i, 0)),
      out_shape=jax.ShapeDtypeStruct((N, k), jnp.float32),
  )(p, b, w)


def _final_body(p_ref, b_ref, o_ref):
  o_ref[...] = (p_ref[0, :, :NCLASS] + p_ref[1, :, :NCLASS] + b_ref[...])


def _tc_final(p, b):
  return pl.pallas_call(
      _final_body,
      grid=(_GRID,),
      in_specs=[
          pl.BlockSpec((2, _ROWS, NHID), lambda i: (0, i, 0)),
          pl.BlockSpec((1, NCLASS), lambda i: (0, 0)),
      ],
      out_specs=pl.BlockSpec((_ROWS, NCLASS), lambda i: (i, 0)),
      out_shape=jax.ShapeDtypeStruct((N, NCLASS), jnp.float32),
  )(p, b)


def _agg(s, src3, dst3, adj):
  return _edge_agg(s, src3, dst3, adj)[:, :N, :]


def kernel(x, adj, edge_index, isVal, W0, b0, W1, b1, W2, b2, Wo, bo):
  del isVal
  src3 = edge_index[0].reshape(32 * _NCHUNK, _CHUNK, _B)
  dst3 = edge_index[1].reshape(32 * _NCHUNK, _CHUNK, _B)
  wo_pad = jnp.pad(Wo, ((0, 0), (0, NHID - NCLASS)))

  s = _tc_matmul(x, W0)
  p = _agg(s, src3, dst3, adj)
  s = _tc_combine_matmul(p, b0.reshape(1, NHID), W1)
  p = _agg(s, src3, dst3, adj)
  s = _tc_combine_matmul(p, b1.reshape(1, NHID), W2)
  p = _agg(s, src3, dst3, adj)
  s = _tc_combine_matmul(p, b2.reshape(1, NHID), wo_pad)
  p = _agg(s, src3, dst3, adj)
  return _tc_final(p, bo.reshape(1, NCLASS))
